# Initial kernel scaffold; baseline (speedup 1.0000x reference)
import jax
import jax.numpy as jnp
from jax.experimental import pallas as pl


def kernel(graph, node_features, W1, b1, W2, b2, Wg, bg):
    raise NotImplementedError("write your pallas kernel here")


# speedup vs baseline: 3.5354x; 3.5354x over previous
import functools

import jax
import jax.numpy as jnp
from jax import lax
from jax.experimental import pallas as pl
from jax.experimental.pallas import tpu as pltpu
from jax.experimental.pallas import tpu_sc as plsc

N = 10000
E = 320000
D = 128
H = 256
C = 64
K = 20

NT = 16
G = 128
CH = (E + NT * G - 1) // (NT * G)
E_PAD = CH * NT * G
NP = 10240
NPT = NP // NT
NB = 64
NSB = NPT // NB

_f32 = jnp.float32
_i32 = jnp.int32

_MESH = plsc.VectorSubcoreMesh(
    core_axis_name="c", subcore_axis_name="s", num_cores=1, num_subcores=NT)


def _unpack_chunk(packed_v, j, sidx_v, didx_v):
  for g in range(G // 16):
    sl = pl.ds(g * 16, 16)
    v = packed_v[j, 0, sl]
    sidx_v[sl] = lax.shift_right_logical(v, 16)
    didx_v[sl] = lax.bitwise_and(v, 0xFFFF)


def _scale_rows(blk_v, d_v, dbase):
  def body(n16, _):
    dvec = d_v[pl.ds(dbase + n16 * 16, 16)]
    for i in range(16):
      dd = dvec[i]
      row = n16 * 16 + i
      for c4 in range(C // 16):
        sl = pl.ds(c4 * 16, 16)
        blk_v[row, sl] = blk_v[row, sl] * dd
    return 0

  lax.fori_loop(0, NB // 16, body, 0, unroll=False)


def _zero_vmem_2d(ref, rows, cols):
  zero16 = jnp.zeros((16,), _f32)

  def body(i, _):
    for c4 in range(cols // 16):
      ref[i, pl.ds(c4 * 16, 16)] = zero16
    return 0

  lax.fori_loop(0, rows, body, 0, unroll=False)


@functools.partial(
    pl.kernel,
    out_type=(jax.ShapeDtypeStruct((NP,), _f32),
              jax.ShapeDtypeStruct((NP,), _f32)),
    mesh=_MESH,
    scratch_types=[
        pltpu.VMEM_SHARED((NP,), _f32),
        pltpu.VMEM_SHARED((NP,), _f32),
        pltpu.VMEM((CH, 1, G), _i32),
        pltpu.VMEM((G,), _i32),
        pltpu.VMEM((G,), _i32),
        pltpu.VMEM((G,), _f32),
        pltpu.VMEM((NPT,), _f32),
    ],
)
def _deg_kernel(packed_hbm, do_hbm, di_hbm,
                do_sh, di_sh, packed_v, sidx_v, didx_v, ones_v, blk_v):
  t = lax.axis_index("s")
  base = t * NPT

  zero16 = jnp.zeros((16,), _f32)
  one16 = jnp.ones((16,), _f32)

  def zb(i, _):
    blk_v[pl.ds(i * 16, 16)] = zero16
    return 0

  lax.fori_loop(0, NPT // 16, zb, 0, unroll=False)
  pltpu.sync_copy(blk_v, do_sh.at[pl.ds(base, NPT)])
  pltpu.sync_copy(blk_v, di_sh.at[pl.ds(base, NPT)])

  pltpu.sync_copy(packed_hbm.at[t], packed_v)
  for g in range(G // 16):
    ones_v[pl.ds(g * 16, 16)] = one16

  plsc.subcore_barrier()

  def chunk(j, _):
    _unpack_chunk(packed_v, j, sidx_v, didx_v)
    pltpu.sync_copy(ones_v, do_sh.at[sidx_v], add=True)
    pltpu.sync_copy(ones_v, di_sh.at[didx_v], add=True)
    return 0

  lax.fori_loop(0, CH, chunk, 0, unroll=False)
  plsc.subcore_barrier()

  pltpu.sync_copy(do_sh.at[pl.ds(base, NPT)], blk_v)
  pltpu.sync_copy(blk_v, do_hbm.at[pl.ds(base, NPT)])
  pltpu.sync_copy(di_sh.at[pl.ds(base, NPT)], blk_v)
  pltpu.sync_copy(blk_v, di_hbm.at[pl.ds(base, NPT)])


_MLP_BN = 2000


def _mlp_body(x_ref, w1_ref, b1_ref, w2_ref, b2_ref, do_ref, di_ref,
              logits_ref, z0_ref, ddst_ref, dsrc_ref):
  h = jnp.maximum(
      jnp.dot(x_ref[...], w1_ref[...], preferred_element_type=_f32)
      + b1_ref[...], 0.0)
  logits = jnp.dot(h, w2_ref[...], preferred_element_type=_f32) + b2_ref[...]
  dsrc = lax.rsqrt(jnp.maximum(do_ref[...], 1.0))
  ddst = lax.rsqrt(jnp.maximum(di_ref[...], 1.0))
  logits_ref[...] = logits
  z0_ref[...] = logits * dsrc
  ddst_ref[...] = ddst
  dsrc_ref[...] = dsrc


def _mlp_call(x, w1, b1, w2, b2, degout, degin):
  grid = (N // _MLP_BN,)
  return pl.pallas_call(
      _mlp_body,
      grid=grid,
      in_specs=[
          pl.BlockSpec((_MLP_BN, D), lambda i: (i, 0)),
          pl.BlockSpec((D, H), lambda i: (0, 0)),
          pl.BlockSpec((1, H), lambda i: (0, 0)),
          pl.BlockSpec((H, C), lambda i: (0, 0)),
          pl.BlockSpec((1, C), lambda i: (0, 0)),
          pl.BlockSpec((_MLP_BN, 1), lambda i: (i, 0)),
          pl.BlockSpec((_MLP_BN, 1), lambda i: (i, 0)),
      ],
      out_specs=[
          pl.BlockSpec((_MLP_BN, C), lambda i: (i, 0)),
          pl.BlockSpec((_MLP_BN, C), lambda i: (i, 0)),
          pl.BlockSpec((_MLP_BN, 1), lambda i: (i, 0)),
          pl.BlockSpec((_MLP_BN, 1), lambda i: (i, 0)),
      ],
      out_shape=[
          jax.ShapeDtypeStruct((N, C), _f32),
          jax.ShapeDtypeStruct((N, C), _f32),
          jax.ShapeDtypeStruct((N, 1), _f32),
          jax.ShapeDtypeStruct((N, 1), _f32),
      ],
  )(x, w1, b1, w2, b2, degout, degin)


@functools.partial(
    pl.kernel,
    out_type=jax.ShapeDtypeStruct((K, NP, C), _f32),
    mesh=_MESH,
    scratch_types=[
        pltpu.VMEM_SHARED((NP, C), _f32),
        pltpu.VMEM_SHARED((NP, C), _f32),
        pltpu.VMEM((CH, 1, G), _i32),
        pltpu.VMEM((G,), _i32),
        pltpu.VMEM((G,), _i32),
        pltpu.VMEM((G, C), _f32),
        pltpu.VMEM((NB, C), _f32),
        pltpu.VMEM((NPT,), _f32),
        pltpu.VMEM((NPT,), _f32),
    ],
)
def _prop_kernel(z0_hbm, ddst_hbm, dsrc_hbm, packed_hbm, out_hbm,
                 z_sh, acc_sh, packed_v, sidx_v, didx_v, rows_v, blk_v,
                 ddst_v, dsrc_v):
  t = lax.axis_index("s")
  base = t * NPT

  pltpu.sync_copy(packed_hbm.at[t], packed_v)
  pltpu.sync_copy(ddst_hbm.at[t], ddst_v)
  pltpu.sync_copy(dsrc_hbm.at[t], dsrc_v)
  for nb in range(NSB):
    sl = pl.ds(base + nb * NB, NB)
    pltpu.sync_copy(z0_hbm.at[sl], blk_v)
    pltpu.sync_copy(blk_v, z_sh.at[sl])
  _zero_vmem_2d(blk_v, NB, C)
  for nb in range(NSB):
    pltpu.sync_copy(blk_v, acc_sh.at[pl.ds(base + nb * NB, NB)])
  plsc.subcore_barrier()

  def step(k, _):
    def chunk(j, _):
      _unpack_chunk(packed_v, j, sidx_v, didx_v)
      pltpu.sync_copy(z_sh.at[sidx_v], rows_v)
      pltpu.sync_copy(rows_v, acc_sh.at[didx_v], add=True)
      return 0

    lax.fori_loop(0, CH, chunk, 0, unroll=False)
    plsc.subcore_barrier()

    for nb in range(NSB):
      sl = pl.ds(base + nb * NB, NB)
      pltpu.sync_copy(acc_sh.at[sl], blk_v)
      _scale_rows(blk_v, ddst_v, nb * NB)
      pltpu.sync_copy(blk_v, out_hbm.at[k, sl])
      _scale_rows(blk_v, dsrc_v, nb * NB)
      pltpu.sync_copy(blk_v, z_sh.at[sl])
      _zero_vmem_2d(blk_v, NB, C)
      pltpu.sync_copy(blk_v, acc_sh.at[sl])
    plsc.subcore_barrier()
    return 0

  lax.fori_loop(0, K, step, 0, unroll=False)


_GS_BN = 1024


def _gated_body(lg_ref, prop_ref, wg_ref, bg_ref, out_ref):
  wg = wg_ref[...]
  bg = bg_ref[...]
  x0 = lg_ref[...]
  s = jax.nn.sigmoid(jnp.dot(x0, wg, preferred_element_type=_f32) + bg)
  acc = x0 * s
  for k in range(K):
    xk = prop_ref[k]
    s = jax.nn.sigmoid(jnp.dot(xk, wg, preferred_element_type=_f32) + bg)
    acc = acc + xk * s
  out_ref[...] = acc


def _gated_call(logits_p, prop, wg, bg):
  grid = (NP // _GS_BN,)
  return pl.pallas_call(
      _gated_body,
      grid=grid,
      in_specs=[
          pl.BlockSpec((_GS_BN, C), lambda i: (i, 0)),
          pl.BlockSpec((K, _GS_BN, C), lambda i: (0, i, 0)),
          pl.BlockSpec((C, 1), lambda i: (0, 0)),
          pl.BlockSpec((1, 1), lambda i: (0, 0)),
      ],
      out_specs=pl.BlockSpec((_GS_BN, C), lambda i: (i, 0)),
      out_shape=jax.ShapeDtypeStruct((NP, C), _f32),
  )(logits_p, prop, wg, bg)


def kernel(graph, node_features, W1, b1, W2, b2, Wg, bg):
  src = graph[0]
  dst = graph[1]
  pad = E_PAD - E
  padv = jnp.full((pad,), N, _i32)
  srcp = jnp.concatenate([src, padv])
  dstp = jnp.concatenate([dst, padv])
  packed = jnp.bitwise_or(
      jnp.left_shift(srcp, 16), dstp).reshape(NT, CH, 1, G)

  degout_p, degin_p = _deg_kernel(packed)
  degout = degout_p[:N, None]
  degin = degin_p[:N, None]

  logits, z0, ddst, dsrc = _mlp_call(
      node_features, W1, b1.reshape(1, H), W2, b2.reshape(1, C),
      degout, degin)

  zpad = jnp.zeros((NP - N, C), _f32)
  dpad = jnp.zeros((NP - N,), _f32)
  z0p = jnp.concatenate([z0, zpad])
  ddst_p = jnp.concatenate([ddst[:, 0], dpad]).reshape(NT, NPT)
  dsrc_p = jnp.concatenate([dsrc[:, 0], dpad]).reshape(NT, NPT)

  prop = _prop_kernel(z0p, ddst_p, dsrc_p, packed)

  logits_p = jnp.concatenate([logits, zpad])
  out_p = _gated_call(logits_p, prop, Wg, bg.reshape(1, 1))
  return out_p[:N]
